# baseline (speedup 1.0000x reference)
import functools
import math

import jax
import jax.numpy as jnp
from jax import lax
from jax.experimental import pallas as pl
from jax.experimental.pallas import tpu as pltpu
from jax.experimental.pallas import tpu_sc as plsc

_B, _S, _D, _N, _SLOTS = 4, 2048, 768, 64, 8
_TS = 1024
_NEG = -1e9

_T = _B * _S
_NW = 32
_TPW = _T // _NW
_CH = 32
_NCH = _TPW // _CH


def _router_body(x_ref, tts_ref, set_ref, wq_ref, desc_ref, bq_ref, temp_ref,
                 w_ref, topk_ref, wp_ref, gidx_ref, wqb_ref):
    b = pl.program_id(0)
    si = pl.program_id(1)

    @pl.when(jnp.logical_and(b == 0, si == 0))
    def _():
        wqb_ref[...] = wq_ref[...].astype(jnp.bfloat16)

    scale = 1.0 / math.sqrt(_D)
    x = x_ref[0].astype(jnp.bfloat16)
    q = jax.lax.dot_general(x, wqb_ref[...], (((1,), (1,)), ((), ())),
                            preferred_element_type=jnp.float32)
    q = q + bq_ref[...]
    qb = q.astype(jnp.bfloat16)
    desc_bf = desc_ref[...].astype(jnp.bfloat16)
    scores = jax.lax.dot_general(qb, desc_bf, (((1,), (1,)), ((), ())),
                                 preferred_element_type=jnp.float32) * scale

    iota_n = jax.lax.broadcasted_iota(jnp.int32, (_TS, _N), 1)
    tts = tts_ref[...]
    mask = jnp.zeros((_TS, _N), dtype=jnp.bool_)
    for k in range(_SLOTS):
        mask = jnp.logical_or(mask, iota_n == tts[:, k][:, None])
    s_masked = jnp.where(mask, scores, _NEG)

    v1 = jnp.max(s_masked, axis=1, keepdims=True)
    i1 = jnp.min(jnp.where(s_masked == v1, iota_n, _N), axis=1)
    hit1 = iota_n == i1[:, None]
    s2 = jnp.where(hit1, -jnp.inf, s_masked)
    v2 = jnp.max(s2, axis=1, keepdims=True)
    i2 = jnp.min(jnp.where(s2 == v2, iota_n, _N), axis=1)
    hit2 = iota_n == i2[:, None]

    temp = jnp.maximum(temp_ref[0, 0], 0.5)
    e2 = jnp.exp((v2 - v1) / temp)
    denom = 1.0 + e2
    w1 = 1.0 / denom
    w2 = e2 / denom
    weights = jnp.where(hit1, w1, 0.0) + jnp.where(hit2, w2, 0.0)

    w_ref[0] = weights
    topk = jnp.concatenate([i1[:, None], i2[:, None]], axis=1)
    topk_ref[0] = topk
    wp_ref[0] = jnp.concatenate([w1, w2], axis=1)
    gidx_ref[0] = topk + b * _N


def _tc_router(token_states, set_states, desc_router, tts, W_q, bq2, temp2):
    grid = (_B, _S // _TS)
    return pl.pallas_call(
        _router_body,
        grid=grid,
        in_specs=[
            pl.BlockSpec((1, _TS, _D), lambda b, s: (b, s, 0)),
            pl.BlockSpec((_TS, _SLOTS), lambda b, s: (s, 0)),
            pl.BlockSpec((1, _N, _D), lambda b, s: (b, 0, 0)),
            pl.BlockSpec((_D, _D), lambda b, s: (0, 0)),
            pl.BlockSpec((_N, _D), lambda b, s: (0, 0)),
            pl.BlockSpec((1, _D), lambda b, s: (0, 0)),
            pl.BlockSpec((1, 1), lambda b, s: (0, 0)),
        ],
        out_specs=[
            pl.BlockSpec((1, _TS, _N), lambda b, s: (b, s, 0)),
            pl.BlockSpec((1, _TS, 2), lambda b, s: (b, s, 0)),
            pl.BlockSpec((1, _TS, 2), lambda b, s: (b, s, 0)),
            pl.BlockSpec((1, _TS, 2), lambda b, s: (b, s, 0)),
        ],
        out_shape=[
            jax.ShapeDtypeStruct((_B, _S, _N), jnp.float32),
            jax.ShapeDtypeStruct((_B, _S, 2), jnp.int32),
            jax.ShapeDtypeStruct((_B, _S, 2), jnp.float32),
            jax.ShapeDtypeStruct((_B, _S, 2), jnp.int32),
        ],
        scratch_shapes=[
            pltpu.VMEM((_D, _D), jnp.bfloat16),
        ],
        compiler_params=pltpu.CompilerParams(
            dimension_semantics=("arbitrary", "arbitrary")),
    )(token_states, tts, set_states, W_q, desc_router, bq2, temp2)


@functools.partial(
    pl.kernel,
    out_type=jax.ShapeDtypeStruct((_T, _D), jnp.float32),
    mesh=plsc.VectorSubcoreMesh(core_axis_name="c", subcore_axis_name="s"),
    scratch_types=[
        pltpu.VMEM((2 * _CH,), jnp.int32),
        pltpu.VMEM((2 * _CH,), jnp.float32),
        pltpu.VMEM((2 * _CH, _D), jnp.float32),
        pltpu.VMEM((_CH, _D), jnp.float32),
        pltpu.SemaphoreType.DMA,
    ],
)
def _sc_combine(set_hbm, gidx_hbm, wp_hbm, out_hbm,
                idx_v, wp_v, rows_v, out_v, sem):
    wid = lax.axis_index("s") * 2 + lax.axis_index("c")
    base = wid * _TPW

    def chunk(c, carry):
        tok0 = base + c * _CH
        pltpu.sync_copy(gidx_hbm.at[pl.ds(tok0 * 2, 2 * _CH)], idx_v)
        pltpu.sync_copy(wp_hbm.at[pl.ds(tok0 * 2, 2 * _CH)], wp_v)
        pltpu.async_copy(set_hbm.at[idx_v], rows_v, sem).wait()

        for t in range(_CH):
            wp16 = wp_v[pl.ds((t // 8) * 16, 16)]
            w1 = wp16[2 * (t % 8)]
            w2 = wp16[2 * (t % 8) + 1]

            def col(j, carry3, t=t, w1=w1, w2=w2):
                a = rows_v[2 * t, pl.ds(j * 16, 16)]
                bb = rows_v[2 * t + 1, pl.ds(j * 16, 16)]
                out_v[t, pl.ds(j * 16, 16)] = a * w1 + bb * w2
                return carry3

            lax.fori_loop(0, _D // 16, col, 0)
        pltpu.sync_copy(out_v, out_hbm.at[pl.ds(tok0, _CH)])
        return carry

    lax.fori_loop(0, _NCH, chunk, 0)


@jax.jit
def kernel(token_states, set_states, desc_router, token_to_sets, W_q, b_q,
           temperature):
    bq2 = b_q.reshape(1, _D)
    temp2 = temperature.reshape(1, 1)
    tts = token_to_sets.astype(jnp.int32)

    weights, topk, wpair, gidx = _tc_router(
        token_states, set_states, desc_router, tts, W_q, bq2, temp2)

    set_flat = set_states.reshape(_B * _N, _D)
    token_repr = _sc_combine(
        set_flat, gidx.reshape(_T * 2), wpair.reshape(_T * 2))
    token_repr = token_repr.reshape(_B, _S, _D)

    bank_indices = topk[:, :, 0]
    return token_repr, bank_indices, weights, topk


# speedup vs baseline: 3.2061x; 3.2061x over previous
import math

import jax
import jax.numpy as jnp
from jax.experimental import pallas as pl
from jax.experimental.pallas import tpu as pltpu

_B, _S, _D, _N, _SLOTS = 4, 2048, 768, 64, 8
_TS = 1024
_NEG = -1e9


def _router_body(x_ref, tts_ref, set_ref, wq_ref, desc_ref, bq_ref, temp_ref,
                 repr_ref, w_ref, topk_ref, wqb_ref):
    b = pl.program_id(0)
    si = pl.program_id(1)

    @pl.when(jnp.logical_and(b == 0, si == 0))
    def _():
        wqb_ref[...] = wq_ref[...].astype(jnp.bfloat16)

    scale = 1.0 / math.sqrt(_D)
    x = x_ref[0].astype(jnp.bfloat16)
    q = jax.lax.dot_general(x, wqb_ref[...], (((1,), (1,)), ((), ())),
                            preferred_element_type=jnp.float32)
    q = q + bq_ref[...]
    qb = q.astype(jnp.bfloat16)
    desc_bf = desc_ref[...].astype(jnp.bfloat16)
    scores = jax.lax.dot_general(qb, desc_bf, (((1,), (1,)), ((), ())),
                                 preferred_element_type=jnp.float32) * scale

    iota_n = jax.lax.broadcasted_iota(jnp.int32, (_TS, _N), 1)
    tts = tts_ref[...]
    mask = jnp.zeros((_TS, _N), dtype=jnp.bool_)
    for k in range(_SLOTS):
        mask = jnp.logical_or(mask, iota_n == tts[:, k][:, None])
    s_masked = jnp.where(mask, scores, _NEG)

    v1 = jnp.max(s_masked, axis=1, keepdims=True)
    i1 = jnp.min(jnp.where(s_masked == v1, iota_n, _N), axis=1)
    hit1 = iota_n == i1[:, None]
    s2 = jnp.where(hit1, -jnp.inf, s_masked)
    v2 = jnp.max(s2, axis=1, keepdims=True)
    i2 = jnp.min(jnp.where(s2 == v2, iota_n, _N), axis=1)
    hit2 = iota_n == i2[:, None]

    temp = jnp.maximum(temp_ref[0, 0], 0.5)
    e2 = jnp.exp((v2 - v1) / temp)
    denom = 1.0 + e2
    w1 = 1.0 / denom
    w2 = e2 / denom
    weights = jnp.where(hit1, w1, 0.0) + jnp.where(hit2, w2, 0.0)

    w_ref[0] = weights
    repr_ref[0] = jax.lax.dot_general(
        weights.astype(jnp.bfloat16), set_ref[0].astype(jnp.bfloat16),
        (((1,), (0,)), ((), ())), preferred_element_type=jnp.float32)
    topk_ref[0] = jnp.concatenate([i1[:, None], i2[:, None]], axis=1)


@jax.jit
def kernel(token_states, set_states, desc_router, token_to_sets, W_q, b_q,
           temperature):
    bq2 = b_q.reshape(1, _D)
    temp2 = temperature.reshape(1, 1)
    tts = token_to_sets.astype(jnp.int32)

    grid = (_B, _S // _TS)
    token_repr, weights, topk = pl.pallas_call(
        _router_body,
        grid=grid,
        in_specs=[
            pl.BlockSpec((1, _TS, _D), lambda b, s: (b, s, 0)),
            pl.BlockSpec((_TS, _SLOTS), lambda b, s: (s, 0)),
            pl.BlockSpec((1, _N, _D), lambda b, s: (b, 0, 0)),
            pl.BlockSpec((_D, _D), lambda b, s: (0, 0)),
            pl.BlockSpec((_N, _D), lambda b, s: (0, 0)),
            pl.BlockSpec((1, _D), lambda b, s: (0, 0)),
            pl.BlockSpec((1, 1), lambda b, s: (0, 0)),
        ],
        out_specs=[
            pl.BlockSpec((1, _TS, _D), lambda b, s: (b, s, 0)),
            pl.BlockSpec((1, _TS, _N), lambda b, s: (b, s, 0)),
            pl.BlockSpec((1, _TS, 2), lambda b, s: (b, s, 0)),
        ],
        out_shape=[
            jax.ShapeDtypeStruct((_B, _S, _D), jnp.float32),
            jax.ShapeDtypeStruct((_B, _S, _N), jnp.float32),
            jax.ShapeDtypeStruct((_B, _S, 2), jnp.int32),
        ],
        scratch_shapes=[
            pltpu.VMEM((_D, _D), jnp.bfloat16),
        ],
        compiler_params=pltpu.CompilerParams(
            dimension_semantics=("arbitrary", "arbitrary")),
    )(token_states, tts, set_states, W_q, desc_router, bq2, temp2)

    bank_indices = topk[:, :, 0]
    return token_repr, bank_indices, weights, topk
